# baseline (speedup 1.0000x reference)
import functools

import jax
import jax.numpy as jnp
from jax import lax
from jax.experimental import pallas as pl
from jax.experimental.pallas import tpu as pltpu
from jax.experimental.pallas import tpu_sc as plsc

ROWS, COLS = 4096, 200
VOCAB = 1000000
D = 64
SCALE = 8.0
NC, NS = 2, 16
NW = NC * NS
K = 128
NIB = ROWS // K
NCHUNK = COLS * NIB
CPW = NCHUNK // NW
NBUF = 4
L = 16


def _emb_body(idx_hbm, table_hbm, out_hbm, idx_v, rows_v, tbuf_v, gsem, osem):
    wid = lax.axis_index("s") * NC + lax.axis_index("c")

    pltpu.sync_copy(idx_hbm.at[wid], idx_v)

    for b in range(NBUF):
        pltpu.async_copy(
            table_hbm.at[idx_v.at[b]], rows_v.at[pl.ds(b * K, K)], gsem.at[b]
        )

    lane = lax.iota(jnp.int32, L)

    def outer(g, carry):
        for b in range(NBUF):
            c = g * NBUF + b
            m = wid * CPW + c
            j = m // NIB
            iblk = m % NIB
            pltpu.make_async_copy(
                table_hbm.at[idx_v.at[0]], rows_v.at[pl.ds(b * K, K)],
                gsem.at[b],
            ).wait()

            @pl.when(g > 0)
            def _():
                pltpu.make_async_copy(
                    tbuf_v.at[pl.ds(b * 8, 8)],
                    out_hbm.at[0, :, 0],
                    osem.at[b],
                ).wait()

            @plsc.parallel_loop(0, K // L, unroll=2)
            def _(tq):
                row_idx = lane + (b * K + tq * L)
                for kb in range(8):
                    for kl in range(8):
                        col = jnp.full((L,), kb * 8 + kl, jnp.int32)
                        vals = plsc.load_gather(rows_v, [row_idx, col])
                        tbuf_v[b * 8 + kb, kl, pl.ds(tq * L, L)] = vals * SCALE

            pltpu.async_copy(
                tbuf_v.at[pl.ds(b * 8, 8)],
                out_hbm.at[j, :, iblk],
                osem.at[b],
            )

            cn = c + NBUF

            @pl.when(cn < CPW)
            def _():
                pltpu.async_copy(
                    table_hbm.at[idx_v.at[cn]], rows_v.at[pl.ds(b * K, K)],
                    gsem.at[b],
                )

        return carry

    lax.fori_loop(0, CPW // NBUF, outer, 0)

    for b in range(NBUF):
        pltpu.make_async_copy(
            tbuf_v.at[pl.ds(b * 8, 8)],
            out_hbm.at[0, :, 0],
            osem.at[b],
        ).wait()


@jax.jit
def _embed(idx, table):
    mesh = plsc.VectorSubcoreMesh(
        core_axis_name="c", subcore_axis_name="s", num_cores=NC, num_subcores=NS
    )
    fn = pl.kernel(
        _emb_body,
        out_type=jax.ShapeDtypeStruct((COLS, D // 8, NIB, 8, K), jnp.float32),
        mesh=mesh,
        compiler_params=pltpu.CompilerParams(use_tc_tiling_on_sc=False, needs_layout_passes=False),
        scratch_types=[
            pltpu.VMEM((CPW, K), jnp.int32),
            pltpu.VMEM((NBUF * K, D), jnp.float32),
            pltpu.VMEM((NBUF * D // 8, 8, K), jnp.float32),
            pltpu.SemaphoreType.DMA((NBUF,)),
            pltpu.SemaphoreType.DMA((NBUF,)),
        ],
    )
    return fn(idx, table)


def kernel(tokens, table):
    idx = tokens.T.reshape(NW, CPW, K)
    out5 = _embed(idx, table)
    return jnp.transpose(out5, (2, 4, 0, 1, 3)).reshape(ROWS, COLS, D)


# speedup vs baseline: 1.9531x; 1.9531x over previous
import functools

import jax
import jax.numpy as jnp
from jax import lax
from jax.experimental import pallas as pl
from jax.experimental.pallas import tpu as pltpu
from jax.experimental.pallas import tpu_sc as plsc

ROWS, COLS = 4096, 200
VOCAB = 1000000
D = 64
SCALE = 8.0
NC, NS = 2, 16
NW = NC * NS
K = 128
NIB = ROWS // K
NCHUNK = COLS * NIB
CPW = NCHUNK // NW
NBUF = 4
L = 16


def _emb_body(idx_hbm, table_hbm, out_hbm, idx_v, rows_v, tbuf_v, gsem, osem):
    wid = lax.axis_index("s") * NC + lax.axis_index("c")

    pltpu.sync_copy(idx_hbm.at[wid], idx_v)

    for b in range(NBUF):
        pltpu.async_copy(
            table_hbm.at[idx_v.at[b]], rows_v.at[pl.ds(b * K, K), pl.ds(0, D)], gsem.at[b]
        )

    lane = lax.iota(jnp.int32, L)
    row_c = [
        [jnp.full((L,), b * 8, jnp.int32) + (lane + kq * L) // 8
         for kq in range(D // L)]
        for b in range(NBUF)
    ]
    kl_c = [lax.rem(lane + kq * L, 8) for kq in range(D // L)]

    def outer(g, carry):
        for b in range(NBUF):
            c = g * NBUF + b
            m = wid * CPW + c
            j = m // NIB
            iblk = m % NIB
            pltpu.make_async_copy(
                table_hbm.at[idx_v.at[0]], rows_v.at[pl.ds(b * K, K), pl.ds(0, D)],
                gsem.at[b],
            ).wait()

            @pl.when(g > 0)
            def _():
                pltpu.make_async_copy(
                    tbuf_v.at[pl.ds(b * 8, 8), :, pl.ds(0, K)],
                    out_hbm.at[0, :, 0],
                    osem.at[b],
                ).wait()

            @plsc.parallel_loop(0, K, unroll=8)
            def _(t):
                t_idx = lane * 0 + t
                for kq in range(D // L):
                    vals = rows_v[b * K + t, pl.ds(kq * L, L)] * SCALE
                    plsc.store_scatter(
                        tbuf_v, [row_c[b][kq], kl_c[kq], t_idx], vals
                    )

            pltpu.async_copy(
                tbuf_v.at[pl.ds(b * 8, 8), :, pl.ds(0, K)],
                out_hbm.at[j, :, iblk],
                osem.at[b],
            )

            cn = c + NBUF

            @pl.when(cn < CPW)
            def _():
                pltpu.async_copy(
                    table_hbm.at[idx_v.at[cn]], rows_v.at[pl.ds(b * K, K), pl.ds(0, D)],
                    gsem.at[b],
                )

        return carry

    lax.fori_loop(0, CPW // NBUF, outer, 0)

    for b in range(NBUF):
        pltpu.make_async_copy(
            tbuf_v.at[pl.ds(b * 8, 8), :, pl.ds(0, K)],
            out_hbm.at[0, :, 0],
            osem.at[b],
        ).wait()


@jax.jit
def _embed(idx, table):
    mesh = plsc.VectorSubcoreMesh(
        core_axis_name="c", subcore_axis_name="s", num_cores=NC, num_subcores=NS
    )
    fn = pl.kernel(
        _emb_body,
        out_type=jax.ShapeDtypeStruct((COLS, D // 8, NIB, 8, K), jnp.float32),
        mesh=mesh,
        compiler_params=pltpu.CompilerParams(use_tc_tiling_on_sc=False, needs_layout_passes=False),
        scratch_types=[
            pltpu.VMEM((CPW, K), jnp.int32),
            pltpu.VMEM((NBUF * K, D), jnp.float32),
            pltpu.VMEM((NBUF * D // 8, 8, K + 1), jnp.float32),
            pltpu.SemaphoreType.DMA((NBUF,)),
            pltpu.SemaphoreType.DMA((NBUF,)),
        ],
    )
    return fn(idx, table)


def kernel(tokens, table):
    idx = tokens.T.reshape(NW, CPW, K)
    out5 = _embed(idx, table)
    return jnp.transpose(out5, (2, 4, 0, 1, 3)).reshape(ROWS, COLS, D)
